# baseline (speedup 1.0000x reference)
import functools

import jax
import jax.numpy as jnp
from jax import lax
from jax.experimental import pallas as pl
from jax.experimental.pallas import tpu as pltpu
from jax.experimental.pallas import tpu_sc as plsc

S = 2048
D = 1024
F = 2048
E = 8
K = 2
EP = 128
BLK = 256
NB = 24
ROWS = NB * BLK
NBT = 16
P = S * K

NC, NS = 2, 16
NW = NC * NS
TPW = S // NW
CHUNK = 32

_NEG = -1e30


RB = 1024


def _router_body(x_ref, rw_ref, rb_ref, probs_ref, meta_ref):
    lT = lax.dot_general(rw_ref[...], x_ref[...], (((0,), (1,)), ((), ())),
                         preferred_element_type=jnp.float32) + rb_ref[...]
    l8 = lT[:E, :]
    row = lax.broadcasted_iota(jnp.int32, (E, RB), 0)
    rowf = row.astype(jnp.float32)

    m = jnp.max(l8, axis=0, keepdims=True)
    ex = jnp.exp(l8 - m)
    probs_ref[...] = ex / jnp.sum(ex, axis=0, keepdims=True)

    m1 = jnp.max(l8, axis=0, keepdims=True)
    i1 = jnp.min(jnp.where(l8 == m1, rowf, EP), axis=0, keepdims=True)
    l2 = jnp.where(rowf == i1, _NEG, l8)
    m2 = jnp.max(l2, axis=0, keepdims=True)
    i2 = jnp.min(jnp.where(l2 == m2, rowf, EP), axis=0, keepdims=True)

    a = jnp.exp(m2 - m1)
    w1 = 1.0 / (1.0 + a)
    w2 = 1.0 - w1
    meta_ref[...] = jnp.where(row == 0, i1,
                    jnp.where(row == 1, i2,
                    jnp.where(row == 2, w1,
                    jnp.where(row == 3, w2, 0.0))))


def _router(flat_x, rw_pad, rb_padT, *, interpret=False):
    return pl.pallas_call(
        _router_body,
        grid=(S // RB,),
        in_specs=[
            pl.BlockSpec((RB, D), lambda i: (i, 0)),
            pl.BlockSpec((D, EP), lambda i: (0, 0)),
            pl.BlockSpec((EP, 1), lambda i: (0, 0)),
        ],
        out_specs=[
            pl.BlockSpec((E, RB), lambda i: (0, i)),
            pl.BlockSpec((E, RB), lambda i: (0, i)),
        ],
        out_shape=[
            jax.ShapeDtypeStruct((E, S), jnp.float32),
            jax.ShapeDtypeStruct((E, S), jnp.float32),
        ],
        interpret=interpret,
    )(flat_x, rw_pad, rb_padT)


def _dispatch_body(e_ref, dest_ref, be_ref):
    jp0 = lax.broadcasted_iota(jnp.int32, (EP, EP), 0)
    jp1 = lax.broadcasted_iota(jnp.int32, (EP, EP), 1)
    e_of = (lax.broadcasted_iota(jnp.int32, (BLK, EP), 1) % E).astype(jnp.float32)

    rr = lax.broadcasted_iota(jnp.int32, (NBT, EP), 0)
    rc = lax.broadcasted_iota(jnp.int32, (NBT, EP), 1)
    rep = (rr == rc // E).astype(jnp.float32)
    e_rep = jnp.dot(e_ref[...], rep, preferred_element_type=jnp.float32)
    oh = (e_rep == e_of).astype(jnp.float32)

    t0 = lax.broadcasted_iota(jnp.int32, (BLK, BLK), 0)
    t1 = lax.broadcasted_iota(jnp.int32, (BLK, BLK), 1)
    tri = (t1 < t0).astype(jnp.float32)
    r_in = jnp.dot(tri, oh, preferred_element_type=jnp.float32)
    ctot = r_in[BLK - 1:BLK, :] + oh[BLK - 1:BLK, :]

    same_e = (jp0 % E == jp1 % E)
    m_bef = (same_e & (jp0 // E < jp1 // E)).astype(jnp.float32)
    before = jnp.dot(ctot, m_bef, preferred_element_type=jnp.float32)
    m_tot = same_e.astype(jnp.float32)
    tot = jnp.dot(ctot, m_tot, preferred_element_type=jnp.float32)
    nblk = jnp.ceil(tot / BLK)
    m_off = ((jp0 // E == 0) & (jp0 % E < jp1 % E)).astype(jnp.float32)
    offb = jnp.dot(nblk, m_off, preferred_element_type=jnp.float32)

    destw = oh * (offb * BLK + before + r_in)
    sg0 = lax.broadcasted_iota(jnp.int32, (EP, NBT), 0)
    sg1 = lax.broadcasted_iota(jnp.int32, (EP, NBT), 1)
    m_sum = (sg0 // E == sg1).astype(jnp.float32)
    dest = jnp.dot(destw, m_sum, preferred_element_type=jnp.float32,
                   precision=lax.Precision.HIGHEST)
    dest_ref[...] = dest.astype(jnp.int32)

    b0 = lax.broadcasted_iota(jnp.int32, (NB + 8, EP), 0).astype(jnp.float32)
    lj = lax.broadcasted_iota(jnp.int32, (NB + 8, EP), 1)
    inb = (b0 >= offb) & (b0 < offb + nblk) & (lj < E)
    ef = (lj % E).astype(jnp.float32)
    vld = jnp.sum(inb.astype(jnp.float32), axis=1, keepdims=True)
    ecol = jnp.sum(jnp.where(inb, ef, 0.0), axis=1, keepdims=True) \
        + (E - 1.0) * (1.0 - vld)
    lz = lax.broadcasted_iota(jnp.int32, (NB + 8, 8), 1)
    be_ref[...] = jnp.where(lz == 0, ecol,
                            jnp.where(lz == 1, vld, 0.0)).astype(jnp.int32)


def _dispatch(e_cm, *, interpret=False):
    return pl.pallas_call(
        _dispatch_body,
        grid=(1,),
        in_specs=[pl.BlockSpec((BLK, NBT), lambda g: (0, 0))],
        out_specs=[
            pl.BlockSpec((BLK, NBT), lambda g: (0, 0)),
            pl.BlockSpec((NB + 8, 8), lambda g: (0, 0)),
        ],
        out_shape=[
            jax.ShapeDtypeStruct((BLK, NBT), jnp.int32),
            jax.ShapeDtypeStruct((NB + 8, 8), jnp.int32),
        ],
        interpret=interpret,
    )(e_cm)


def _scatter_sc(flat_x, dest, wpair):
    mesh = plsc.VectorSubcoreMesh(core_axis_name="c", subcore_axis_name="s")

    H = TPW // 2

    @functools.partial(
        pl.kernel,
        out_type=(jax.ShapeDtypeStruct((ROWS, D), jnp.float32),
                  jax.ShapeDtypeStruct((ROWS,), jnp.float32)),
        mesh=mesh,
        scratch_types=[
            pltpu.VMEM((H,), jnp.int32),
            pltpu.VMEM((H,), jnp.int32),
            pltpu.VMEM((H,), jnp.int32),
            pltpu.VMEM((H,), jnp.int32),
            pltpu.VMEM((TPW,), jnp.float32),
            pltpu.VMEM((TPW,), jnp.float32),
            pltpu.VMEM((H, D), jnp.float32),
            pltpu.VMEM((H, D), jnp.float32),
            pltpu.SemaphoreType.DMA,
            pltpu.SemaphoreType.DMA,
            pltpu.SemaphoreType.DMA,
        ],
    )
    def k(x_hbm, dest_hbm, w_hbm, xs_hbm, wrow_hbm,
          i0a, i0b, i1a, i1b, w0_v, w1_v, ra, rb, semA, semB, semS):
        wid = lax.axis_index("s") * NC + lax.axis_index("c")
        base = wid * TPW
        la = [pltpu.async_copy(dest_hbm.at[pl.ds(base, H)], i0a, semA),
              pltpu.async_copy(dest_hbm.at[pl.ds(S + base, H)], i1a, semA),
              pltpu.async_copy(w_hbm.at[pl.ds(base, TPW)], w0_v, semA),
              pltpu.async_copy(w_hbm.at[pl.ds(S + base, TPW)], w1_v, semA),
              pltpu.async_copy(x_hbm.at[pl.ds(base, H)], ra, semA)]
        lb = [pltpu.async_copy(dest_hbm.at[pl.ds(base + H, H)], i0b, semB),
              pltpu.async_copy(dest_hbm.at[pl.ds(S + base + H, H)], i1b, semB),
              pltpu.async_copy(x_hbm.at[pl.ds(base + H, H)], rb, semB)]
        for c in la:
            c.wait()
        sa = [pltpu.async_copy(ra, xs_hbm.at[i0a], semS),
              pltpu.async_copy(ra, xs_hbm.at[i1a], semS),
              pltpu.async_copy(w0_v.at[pl.ds(0, H)], wrow_hbm.at[i0a], semS),
              pltpu.async_copy(w1_v.at[pl.ds(0, H)], wrow_hbm.at[i1a], semS)]
        for c in lb:
            c.wait()
        sb = [pltpu.async_copy(rb, xs_hbm.at[i0b], semS),
              pltpu.async_copy(rb, xs_hbm.at[i1b], semS),
              pltpu.async_copy(w0_v.at[pl.ds(H, H)], wrow_hbm.at[i0b], semS),
              pltpu.async_copy(w1_v.at[pl.ds(H, H)], wrow_hbm.at[i1b], semS)]
        for c in sa + sb:
            c.wait()

    return k(flat_x, dest, wpair)


NF = 4
FC = F // NF


def _ffn_body(be_ref, vld_ref, xs_ref, wg_ref, bg_ref, wu_ref, bu_ref,
              wd_ref, bd_ref, wrow_ref, ys_ref):
    b = pl.program_id(0)
    f = pl.program_id(1)

    @pl.when(vld_ref[b] == 1)
    def _compute():
        x = xs_ref[...]
        g = jnp.dot(x, wg_ref[0], preferred_element_type=jnp.float32) + bg_ref[0]
        u = jnp.dot(x, wu_ref[0], preferred_element_type=jnp.float32) + bu_ref[0]
        h = g * jax.nn.sigmoid(g) * u
        y = jnp.dot(h, wd_ref[0], preferred_element_type=jnp.float32)

        @pl.when(f == 0)
        def _init():
            ys_ref[...] = (y + bd_ref[0]) * wrow_ref[0]

        @pl.when(f > 0)
        def _acc():
            ys_ref[...] = ys_ref[...] + y * wrow_ref[0]


def _ffn(xs, wg, bg, wu, bu, wd, bd, wrow3, be, vld, *, interpret=False):
    grid_spec = pltpu.PrefetchScalarGridSpec(
        num_scalar_prefetch=2,
        grid=(NB, NF),
        in_specs=[
            pl.BlockSpec((BLK, D), lambda b, f, be, vld: (b, 0)),
            pl.BlockSpec((1, D, FC), lambda b, f, be, vld: (be[b], 0, f)),
            pl.BlockSpec((1, 1, FC), lambda b, f, be, vld: (be[b], 0, f)),
            pl.BlockSpec((1, D, FC), lambda b, f, be, vld: (be[b], 0, f)),
            pl.BlockSpec((1, 1, FC), lambda b, f, be, vld: (be[b], 0, f)),
            pl.BlockSpec((1, FC, D), lambda b, f, be, vld: (be[b], f, 0)),
            pl.BlockSpec((1, 1, D), lambda b, f, be, vld: (be[b], 0, 0)),
            pl.BlockSpec((1, BLK, 1), lambda b, f, be, vld: (b, 0, 0)),
        ],
        out_specs=pl.BlockSpec((BLK, D), lambda b, f, be, vld: (b, 0)),
    )
    return pl.pallas_call(
        _ffn_body,
        grid_spec=grid_spec,
        out_shape=jax.ShapeDtypeStruct((ROWS, D), jnp.float32),
        interpret=interpret,
    )(be, vld, xs, wg, bg, wu, bu, wd, bd, wrow3)


def _combine_sc(ys, dest):
    mesh = plsc.VectorSubcoreMesh(core_axis_name="c", subcore_axis_name="s")
    nch = TPW // CHUNK

    @functools.partial(
        pl.kernel,
        out_type=jax.ShapeDtypeStruct((S, D), jnp.float32),
        mesh=mesh,
        scratch_types=[
            pltpu.VMEM((CHUNK,), jnp.int32),
            pltpu.VMEM((CHUNK,), jnp.int32),
            pltpu.VMEM((CHUNK, D), jnp.float32),
            pltpu.VMEM((CHUNK, D), jnp.float32),
            pltpu.VMEM((CHUNK, D), jnp.float32),
            pltpu.SemaphoreType.DMA,
        ],
    )
    def k(ys_hbm, dest_hbm, out_hbm, i0_v, i1_v, a_v, b_v, o_v, sem):
        wid = lax.axis_index("s") * NC + lax.axis_index("c")
        base = wid * TPW

        def chunk(c, _):
            cb = base + c * CHUNK
            pltpu.sync_copy(dest_hbm.at[pl.ds(cb, CHUNK)], i0_v)
            pltpu.sync_copy(dest_hbm.at[pl.ds(S + cb, CHUNK)], i1_v)
            g0 = pltpu.async_copy(ys_hbm.at[i0_v], a_v, sem)
            g1 = pltpu.async_copy(ys_hbm.at[i1_v], b_v, sem)
            g0.wait()
            g1.wait()

            def tok(i, _):
                for v in range(D // 16):
                    sl = pl.ds(v * 16, 16)
                    o_v[i, sl] = a_v[i, sl] + b_v[i, sl]
                return 0

            lax.fori_loop(0, CHUNK, tok, 0)
            pltpu.sync_copy(o_v, out_hbm.at[pl.ds(cb, CHUNK)])
            return 0

        lax.fori_loop(0, nch, chunk, 0)

    return k(ys, dest)


def kernel(x, router_w, router_b, wg, bg, wu, bu, wd, bd):
    bsz, seq, _ = x.shape
    flat_x = x.reshape(S, D)
    rw_pad = jnp.zeros((D, EP), jnp.float32).at[:, :E].set(router_w)
    rb_padT = jnp.zeros((EP, 1), jnp.float32).at[:E, 0].set(router_b)

    probsT, meta = _router(flat_x, rw_pad, rb_padT)
    i1 = meta[0].astype(jnp.int32)
    i2 = meta[1].astype(jnp.int32)

    e_cm = jnp.concatenate([meta[0].reshape(E, BLK).T,
                            meta[1].reshape(E, BLK).T], axis=1)
    dest2d, bev = _dispatch(e_cm)
    dest = dest2d.T.reshape(P)
    be = bev[:NB, 0]
    vld = bev[:NB, 1]

    wpair = jnp.concatenate([meta[2], meta[3]])
    xs, wrow = _scatter_sc(flat_x, dest, wpair)
    ys = _ffn(xs, wg, bg.reshape(E, 1, F), wu, bu.reshape(E, 1, F),
              wd, bd.reshape(E, 1, D), wrow.reshape(NB, BLK, 1), be, vld)
    out = _combine_sc(ys, dest)

    return (out.reshape(bsz, seq, D),
            probsT.T.reshape(bsz, seq, E),
            jnp.stack([i1, i2], axis=1).reshape(bsz, seq, K))


# speedup vs baseline: 1.4025x; 1.4025x over previous
import functools

import jax
import jax.numpy as jnp
from jax import lax
from jax.experimental import pallas as pl
from jax.experimental.pallas import tpu as pltpu
from jax.experimental.pallas import tpu_sc as plsc

S = 2048
D = 1024
F = 2048
E = 8
K = 2
EP = 128
BLK = 256
NB = 24
ROWS = NB * BLK
NBT = 16
P = S * K

NC, NS = 2, 16
NW = NC * NS
TPW = S // NW
CHUNK = 32

_NEG = -1e30


RB = 1024


def _router_body(x_ref, rw_ref, rb_ref, probs_ref, meta_ref):
    lT = lax.dot_general(rw_ref[...], x_ref[...], (((0,), (1,)), ((), ())),
                         preferred_element_type=jnp.float32) + rb_ref[...]
    l8 = lT[:E, :]
    row = lax.broadcasted_iota(jnp.int32, (E, RB), 0)
    rowf = row.astype(jnp.float32)

    m = jnp.max(l8, axis=0, keepdims=True)
    ex = jnp.exp(l8 - m)
    probs_ref[...] = ex / jnp.sum(ex, axis=0, keepdims=True)

    m1 = jnp.max(l8, axis=0, keepdims=True)
    i1 = jnp.min(jnp.where(l8 == m1, rowf, EP), axis=0, keepdims=True)
    l2 = jnp.where(rowf == i1, _NEG, l8)
    m2 = jnp.max(l2, axis=0, keepdims=True)
    i2 = jnp.min(jnp.where(l2 == m2, rowf, EP), axis=0, keepdims=True)

    a = jnp.exp(m2 - m1)
    w1 = 1.0 / (1.0 + a)
    w2 = 1.0 - w1
    meta_ref[...] = jnp.where(row == 0, i1,
                    jnp.where(row == 1, i2,
                    jnp.where(row == 2, w1,
                    jnp.where(row == 3, w2, 0.0))))


def _router(flat_x, rw_pad, rb_padT, *, interpret=False):
    return pl.pallas_call(
        _router_body,
        grid=(S // RB,),
        in_specs=[
            pl.BlockSpec((RB, D), lambda i: (i, 0)),
            pl.BlockSpec((D, EP), lambda i: (0, 0)),
            pl.BlockSpec((EP, 1), lambda i: (0, 0)),
        ],
        out_specs=[
            pl.BlockSpec((E, RB), lambda i: (0, i)),
            pl.BlockSpec((E, RB), lambda i: (0, i)),
        ],
        out_shape=[
            jax.ShapeDtypeStruct((E, S), jnp.float32),
            jax.ShapeDtypeStruct((E, S), jnp.float32),
        ],
        interpret=interpret,
    )(flat_x, rw_pad, rb_padT)


def _dispatch_body(e_ref, dest_ref, be_ref):
    jp0 = lax.broadcasted_iota(jnp.int32, (EP, EP), 0)
    jp1 = lax.broadcasted_iota(jnp.int32, (EP, EP), 1)
    e_of = (lax.broadcasted_iota(jnp.int32, (BLK, EP), 1) % E).astype(jnp.float32)

    rr = lax.broadcasted_iota(jnp.int32, (NBT, EP), 0)
    rc = lax.broadcasted_iota(jnp.int32, (NBT, EP), 1)
    rep = (rr == rc // E).astype(jnp.float32)
    e_rep = jnp.dot(e_ref[...], rep, preferred_element_type=jnp.float32)
    oh = (e_rep == e_of).astype(jnp.float32)

    t0 = lax.broadcasted_iota(jnp.int32, (BLK, BLK), 0)
    t1 = lax.broadcasted_iota(jnp.int32, (BLK, BLK), 1)
    tri = (t1 < t0).astype(jnp.float32)
    r_in = jnp.dot(tri, oh, preferred_element_type=jnp.float32)
    ctot = r_in[BLK - 1:BLK, :] + oh[BLK - 1:BLK, :]

    same_e = (jp0 % E == jp1 % E)
    m_bef = (same_e & (jp0 // E < jp1 // E)).astype(jnp.float32)
    before = jnp.dot(ctot, m_bef, preferred_element_type=jnp.float32)
    m_tot = same_e.astype(jnp.float32)
    tot = jnp.dot(ctot, m_tot, preferred_element_type=jnp.float32)
    nblk = jnp.ceil(tot / BLK)
    m_off = ((jp0 // E == 0) & (jp0 % E < jp1 % E)).astype(jnp.float32)
    offb = jnp.dot(nblk, m_off, preferred_element_type=jnp.float32)

    destw = oh * (offb * BLK + before + r_in)
    sg0 = lax.broadcasted_iota(jnp.int32, (EP, NBT), 0)
    sg1 = lax.broadcasted_iota(jnp.int32, (EP, NBT), 1)
    m_sum = (sg0 // E == sg1).astype(jnp.float32)
    dest = jnp.dot(destw, m_sum, preferred_element_type=jnp.float32,
                   precision=lax.Precision.HIGHEST)
    dest_ref[...] = dest.astype(jnp.int32)

    b0 = lax.broadcasted_iota(jnp.int32, (NB + 8, EP), 0).astype(jnp.float32)
    lj = lax.broadcasted_iota(jnp.int32, (NB + 8, EP), 1)
    inb = (b0 >= offb) & (b0 < offb + nblk) & (lj < E)
    ef = (lj % E).astype(jnp.float32)
    vld = jnp.sum(inb.astype(jnp.float32), axis=1, keepdims=True)
    ecol = jnp.sum(jnp.where(inb, ef, 0.0), axis=1, keepdims=True) \
        + (E - 1.0) * (1.0 - vld)
    lz = lax.broadcasted_iota(jnp.int32, (NB + 8, 8), 1)
    be_ref[...] = jnp.where(lz == 0, ecol,
                            jnp.where(lz == 1, vld, 0.0)).astype(jnp.int32)


def _dispatch(e_cm, *, interpret=False):
    return pl.pallas_call(
        _dispatch_body,
        grid=(1,),
        in_specs=[pl.BlockSpec((BLK, NBT), lambda g: (0, 0))],
        out_specs=[
            pl.BlockSpec((BLK, NBT), lambda g: (0, 0)),
            pl.BlockSpec((NB + 8, 8), lambda g: (0, 0)),
        ],
        out_shape=[
            jax.ShapeDtypeStruct((BLK, NBT), jnp.int32),
            jax.ShapeDtypeStruct((NB + 8, 8), jnp.int32),
        ],
        interpret=interpret,
    )(e_cm)


def _scatter_sc(flat_x, dest, wpair):
    mesh = plsc.VectorSubcoreMesh(core_axis_name="c", subcore_axis_name="s")

    H = TPW // 2

    @functools.partial(
        pl.kernel,
        out_type=(jax.ShapeDtypeStruct((ROWS, D), jnp.float32),
                  jax.ShapeDtypeStruct((ROWS,), jnp.float32)),
        mesh=mesh,
        scratch_types=[
            pltpu.VMEM((H,), jnp.int32),
            pltpu.VMEM((H,), jnp.int32),
            pltpu.VMEM((H,), jnp.int32),
            pltpu.VMEM((H,), jnp.int32),
            pltpu.VMEM((TPW,), jnp.float32),
            pltpu.VMEM((TPW,), jnp.float32),
            pltpu.VMEM((H, D), jnp.float32),
            pltpu.VMEM((H, D), jnp.float32),
            pltpu.SemaphoreType.DMA,
            pltpu.SemaphoreType.DMA,
            pltpu.SemaphoreType.DMA,
        ],
    )
    def k(x_hbm, dest_hbm, w_hbm, xs_hbm, wrow_hbm,
          i0a, i0b, i1a, i1b, w0_v, w1_v, ra, rb, semA, semB, semS):
        wid = lax.axis_index("s") * NC + lax.axis_index("c")
        base = wid * TPW
        la = [pltpu.async_copy(dest_hbm.at[pl.ds(base, H)], i0a, semA),
              pltpu.async_copy(dest_hbm.at[pl.ds(S + base, H)], i1a, semA),
              pltpu.async_copy(w_hbm.at[pl.ds(base, TPW)], w0_v, semA),
              pltpu.async_copy(w_hbm.at[pl.ds(S + base, TPW)], w1_v, semA),
              pltpu.async_copy(x_hbm.at[pl.ds(base, H)], ra, semA)]
        lb = [pltpu.async_copy(dest_hbm.at[pl.ds(base + H, H)], i0b, semB),
              pltpu.async_copy(dest_hbm.at[pl.ds(S + base + H, H)], i1b, semB),
              pltpu.async_copy(x_hbm.at[pl.ds(base + H, H)], rb, semB)]
        for c in la:
            c.wait()
        sa = [pltpu.async_copy(ra, xs_hbm.at[i0a], semS),
              pltpu.async_copy(ra, xs_hbm.at[i1a], semS),
              pltpu.async_copy(w0_v.at[pl.ds(0, H)], wrow_hbm.at[i0a], semS),
              pltpu.async_copy(w1_v.at[pl.ds(0, H)], wrow_hbm.at[i1a], semS)]
        for c in lb:
            c.wait()
        sb = [pltpu.async_copy(rb, xs_hbm.at[i0b], semS),
              pltpu.async_copy(rb, xs_hbm.at[i1b], semS),
              pltpu.async_copy(w0_v.at[pl.ds(H, H)], wrow_hbm.at[i0b], semS),
              pltpu.async_copy(w1_v.at[pl.ds(H, H)], wrow_hbm.at[i1b], semS)]
        for c in sa + sb:
            c.wait()

    return k(flat_x, dest, wpair)


def _ffn_body(be_ref, vld_ref, xs_ref, wg_ref, bg_ref, wu_ref, bu_ref,
              wd_ref, bd_ref, wrow_ref, ys_ref):
    b = pl.program_id(0)

    @pl.when(vld_ref[b] == 1)
    def _compute():
        x = xs_ref[...]
        g = jnp.dot(x, wg_ref[0], preferred_element_type=jnp.float32) + bg_ref[0]
        u = jnp.dot(x, wu_ref[0], preferred_element_type=jnp.float32) + bu_ref[0]
        h = g * jax.nn.sigmoid(g) * u
        y = jnp.dot(h, wd_ref[0], preferred_element_type=jnp.float32) + bd_ref[0]
        ys_ref[...] = y * wrow_ref[0]


def _ffn(xs, wg, bg, wu, bu, wd, bd, wrow3, be, vld, *, interpret=False):
    grid_spec = pltpu.PrefetchScalarGridSpec(
        num_scalar_prefetch=2,
        grid=(NB,),
        in_specs=[
            pl.BlockSpec((BLK, D), lambda b, be, vld: (b, 0)),
            pl.BlockSpec((1, D, F), lambda b, be, vld: (be[b], 0, 0)),
            pl.BlockSpec((1, 1, F), lambda b, be, vld: (be[b], 0, 0)),
            pl.BlockSpec((1, D, F), lambda b, be, vld: (be[b], 0, 0)),
            pl.BlockSpec((1, 1, F), lambda b, be, vld: (be[b], 0, 0)),
            pl.BlockSpec((1, F, D), lambda b, be, vld: (be[b], 0, 0)),
            pl.BlockSpec((1, 1, D), lambda b, be, vld: (be[b], 0, 0)),
            pl.BlockSpec((1, BLK, 1), lambda b, be, vld: (b, 0, 0)),
        ],
        out_specs=pl.BlockSpec((BLK, D), lambda b, be, vld: (b, 0)),
    )
    return pl.pallas_call(
        _ffn_body,
        grid_spec=grid_spec,
        out_shape=jax.ShapeDtypeStruct((ROWS, D), jnp.float32),
        interpret=interpret,
    )(be, vld, xs, wg, bg, wu, bu, wd, bd, wrow3)


def _combine_sc(ys, dest):
    mesh = plsc.VectorSubcoreMesh(core_axis_name="c", subcore_axis_name="s")
    nch = TPW // CHUNK

    @functools.partial(
        pl.kernel,
        out_type=jax.ShapeDtypeStruct((S, D), jnp.float32),
        mesh=mesh,
        scratch_types=[
            pltpu.VMEM((CHUNK,), jnp.int32),
            pltpu.VMEM((CHUNK,), jnp.int32),
            pltpu.VMEM((CHUNK, D), jnp.float32),
            pltpu.VMEM((CHUNK, D), jnp.float32),
            pltpu.VMEM((CHUNK, D), jnp.float32),
            pltpu.SemaphoreType.DMA,
        ],
    )
    def k(ys_hbm, dest_hbm, out_hbm, i0_v, i1_v, a_v, b_v, o_v, sem):
        wid = lax.axis_index("s") * NC + lax.axis_index("c")
        base = wid * TPW

        def chunk(c, _):
            cb = base + c * CHUNK
            pltpu.sync_copy(dest_hbm.at[pl.ds(cb, CHUNK)], i0_v)
            pltpu.sync_copy(dest_hbm.at[pl.ds(S + cb, CHUNK)], i1_v)
            g0 = pltpu.async_copy(ys_hbm.at[i0_v], a_v, sem)
            g1 = pltpu.async_copy(ys_hbm.at[i1_v], b_v, sem)
            g0.wait()
            g1.wait()

            def tok(i, _):
                for v in range(D // 16):
                    sl = pl.ds(v * 16, 16)
                    o_v[i, sl] = a_v[i, sl] + b_v[i, sl]
                return 0

            lax.fori_loop(0, CHUNK, tok, 0)
            pltpu.sync_copy(o_v, out_hbm.at[pl.ds(cb, CHUNK)])
            return 0

        lax.fori_loop(0, nch, chunk, 0)

    return k(ys, dest)


def kernel(x, router_w, router_b, wg, bg, wu, bu, wd, bd):
    bsz, seq, _ = x.shape
    flat_x = x.reshape(S, D)
    rw_pad = jnp.zeros((D, EP), jnp.float32).at[:, :E].set(router_w)
    rb_padT = jnp.zeros((EP, 1), jnp.float32).at[:E, 0].set(router_b)

    probsT, meta = _router(flat_x, rw_pad, rb_padT)
    i1 = meta[0].astype(jnp.int32)
    i2 = meta[1].astype(jnp.int32)

    e_cm = jnp.concatenate([meta[0].reshape(E, BLK).T,
                            meta[1].reshape(E, BLK).T], axis=1)
    dest2d, bev = _dispatch(e_cm)
    dest = dest2d.T.reshape(P)
    be = bev[:NB, 0]
    vld = bev[:NB, 1]

    wpair = jnp.concatenate([meta[2], meta[3]])
    xs, wrow = _scatter_sc(flat_x, dest, wpair)
    ys = _ffn(xs, wg, bg.reshape(E, 1, F), wu, bu.reshape(E, 1, F),
              wd, bd.reshape(E, 1, D), wrow.reshape(NB, BLK, 1), be, vld)
    out = _combine_sc(ys, dest)

    return (out.reshape(bsz, seq, D),
            probsT.T.reshape(bsz, seq, E),
            jnp.stack([i1, i2], axis=1).reshape(bsz, seq, K))


# speedup vs baseline: 1.4356x; 1.0236x over previous
import functools

import jax
import jax.numpy as jnp
from jax import lax
from jax.experimental import pallas as pl
from jax.experimental.pallas import tpu as pltpu
from jax.experimental.pallas import tpu_sc as plsc

S = 2048
D = 1024
F = 2048
E = 8
K = 2
EP = 128
BLK = 256
NB = 24
ROWS = NB * BLK
NBT = 16
P = S * K

NC, NS = 2, 16
NW = NC * NS
TPW = S // NW
CHUNK = 32

_NEG = -1e30


RB = 1024


def _router_body(x_ref, rw_ref, rb_ref, probs_ref, meta_ref):
    lT = lax.dot_general(rw_ref[...], x_ref[...], (((0,), (1,)), ((), ())),
                         preferred_element_type=jnp.float32) + rb_ref[...]
    l8 = lT[:E, :]
    row = lax.broadcasted_iota(jnp.int32, (E, RB), 0)
    rowf = row.astype(jnp.float32)

    m = jnp.max(l8, axis=0, keepdims=True)
    ex = jnp.exp(l8 - m)
    probs_ref[...] = ex / jnp.sum(ex, axis=0, keepdims=True)

    m1 = jnp.max(l8, axis=0, keepdims=True)
    i1 = jnp.min(jnp.where(l8 == m1, rowf, EP), axis=0, keepdims=True)
    l2 = jnp.where(rowf == i1, _NEG, l8)
    m2 = jnp.max(l2, axis=0, keepdims=True)
    i2 = jnp.min(jnp.where(l2 == m2, rowf, EP), axis=0, keepdims=True)

    a = jnp.exp(m2 - m1)
    w1 = 1.0 / (1.0 + a)
    w2 = 1.0 - w1
    meta_ref[...] = jnp.where(row == 0, i1,
                    jnp.where(row == 1, i2,
                    jnp.where(row == 2, w1,
                    jnp.where(row == 3, w2, 0.0))))


def _router(flat_x, rw_pad, rb_padT, *, interpret=False):
    return pl.pallas_call(
        _router_body,
        grid=(S // RB,),
        in_specs=[
            pl.BlockSpec((RB, D), lambda i: (i, 0)),
            pl.BlockSpec((D, EP), lambda i: (0, 0)),
            pl.BlockSpec((EP, 1), lambda i: (0, 0)),
        ],
        out_specs=[
            pl.BlockSpec((E, RB), lambda i: (0, i)),
            pl.BlockSpec((E, RB), lambda i: (0, i)),
        ],
        out_shape=[
            jax.ShapeDtypeStruct((E, S), jnp.float32),
            jax.ShapeDtypeStruct((E, S), jnp.float32),
        ],
        interpret=interpret,
    )(flat_x, rw_pad, rb_padT)


def _dispatch_body(e_ref, dest_ref, be_ref):
    jp0 = lax.broadcasted_iota(jnp.int32, (EP, EP), 0)
    jp1 = lax.broadcasted_iota(jnp.int32, (EP, EP), 1)
    e_of = (lax.broadcasted_iota(jnp.int32, (BLK, EP), 1) % E).astype(jnp.float32)

    rr = lax.broadcasted_iota(jnp.int32, (NBT, EP), 0)
    rc = lax.broadcasted_iota(jnp.int32, (NBT, EP), 1)
    rep = (rr == rc // E).astype(jnp.float32)
    e_rep = jnp.dot(e_ref[...], rep, preferred_element_type=jnp.float32)
    oh = (e_rep == e_of).astype(jnp.float32)

    t0 = lax.broadcasted_iota(jnp.int32, (BLK, BLK), 0)
    t1 = lax.broadcasted_iota(jnp.int32, (BLK, BLK), 1)
    tri = (t1 < t0).astype(jnp.float32)
    r_in = jnp.dot(tri, oh, preferred_element_type=jnp.float32)
    ctot = r_in[BLK - 1:BLK, :] + oh[BLK - 1:BLK, :]

    same_e = (jp0 % E == jp1 % E)
    m_bef = (same_e & (jp0 // E < jp1 // E)).astype(jnp.float32)
    before = jnp.dot(ctot, m_bef, preferred_element_type=jnp.float32)
    m_tot = same_e.astype(jnp.float32)
    tot = jnp.dot(ctot, m_tot, preferred_element_type=jnp.float32)
    nblk = jnp.ceil(tot / BLK)
    m_off = ((jp0 // E == 0) & (jp0 % E < jp1 % E)).astype(jnp.float32)
    offb = jnp.dot(nblk, m_off, preferred_element_type=jnp.float32)

    destw = oh * (offb * BLK + before + r_in)
    sg0 = lax.broadcasted_iota(jnp.int32, (EP, NBT), 0)
    sg1 = lax.broadcasted_iota(jnp.int32, (EP, NBT), 1)
    m_sum = (sg0 // E == sg1).astype(jnp.float32)
    dest = jnp.dot(destw, m_sum, preferred_element_type=jnp.float32,
                   precision=lax.Precision.HIGHEST)
    dest_ref[...] = dest.astype(jnp.int32)

    b0 = lax.broadcasted_iota(jnp.int32, (NB + 8, EP), 0).astype(jnp.float32)
    lj = lax.broadcasted_iota(jnp.int32, (NB + 8, EP), 1)
    inb = (b0 >= offb) & (b0 < offb + nblk) & (lj < E)
    ef = (lj % E).astype(jnp.float32)
    vld = jnp.sum(inb.astype(jnp.float32), axis=1, keepdims=True)
    ecol = jnp.sum(jnp.where(inb, ef, 0.0), axis=1, keepdims=True) \
        + (E - 1.0) * (1.0 - vld)
    lz = lax.broadcasted_iota(jnp.int32, (NB + 8, 8), 1)
    be_ref[...] = jnp.where(lz == 0, ecol,
                            jnp.where(lz == 1, vld, 0.0)).astype(jnp.int32)


def _dispatch(e_cm, *, interpret=False):
    return pl.pallas_call(
        _dispatch_body,
        grid=(1,),
        in_specs=[pl.BlockSpec((BLK, NBT), lambda g: (0, 0))],
        out_specs=[
            pl.BlockSpec((BLK, NBT), lambda g: (0, 0)),
            pl.BlockSpec((NB + 8, 8), lambda g: (0, 0)),
        ],
        out_shape=[
            jax.ShapeDtypeStruct((BLK, NBT), jnp.int32),
            jax.ShapeDtypeStruct((NB + 8, 8), jnp.int32),
        ],
        interpret=interpret,
    )(e_cm)


def _scatter_sc(flat_x, dest, wpair):
    mesh = plsc.VectorSubcoreMesh(core_axis_name="c", subcore_axis_name="s")

    H = TPW // 2

    @functools.partial(
        pl.kernel,
        out_type=(jax.ShapeDtypeStruct((ROWS, D), jnp.float32),
                  jax.ShapeDtypeStruct((ROWS,), jnp.float32)),
        mesh=mesh,
        scratch_types=[
            pltpu.VMEM((H,), jnp.int32),
            pltpu.VMEM((H,), jnp.int32),
            pltpu.VMEM((H,), jnp.int32),
            pltpu.VMEM((H,), jnp.int32),
            pltpu.VMEM((TPW,), jnp.float32),
            pltpu.VMEM((TPW,), jnp.float32),
            pltpu.VMEM((H, D), jnp.float32),
            pltpu.VMEM((H, D), jnp.float32),
            pltpu.SemaphoreType.DMA,
            pltpu.SemaphoreType.DMA,
            pltpu.SemaphoreType.DMA,
        ],
    )
    def k(x_hbm, dest_hbm, w_hbm, xs_hbm, wrow_hbm,
          i0a, i0b, i1a, i1b, w0_v, w1_v, ra, rb, semA, semB, semS):
        wid = lax.axis_index("s") * NC + lax.axis_index("c")
        base = wid * TPW
        la = [pltpu.async_copy(dest_hbm.at[pl.ds(base, H)], i0a, semA),
              pltpu.async_copy(dest_hbm.at[pl.ds(S + base, H)], i1a, semA),
              pltpu.async_copy(w_hbm.at[pl.ds(base, TPW)], w0_v, semA),
              pltpu.async_copy(w_hbm.at[pl.ds(S + base, TPW)], w1_v, semA),
              pltpu.async_copy(x_hbm.at[pl.ds(base, H)], ra, semA)]
        lb = [pltpu.async_copy(dest_hbm.at[pl.ds(base + H, H)], i0b, semB),
              pltpu.async_copy(dest_hbm.at[pl.ds(S + base + H, H)], i1b, semB),
              pltpu.async_copy(x_hbm.at[pl.ds(base + H, H)], rb, semB)]
        for c in la:
            c.wait()
        sa = [pltpu.async_copy(ra, xs_hbm.at[i0a], semS),
              pltpu.async_copy(ra, xs_hbm.at[i1a], semS),
              pltpu.async_copy(w0_v.at[pl.ds(0, H)], wrow_hbm.at[i0a], semS),
              pltpu.async_copy(w1_v.at[pl.ds(0, H)], wrow_hbm.at[i1a], semS)]
        for c in lb:
            c.wait()
        sb = [pltpu.async_copy(rb, xs_hbm.at[i0b], semS),
              pltpu.async_copy(rb, xs_hbm.at[i1b], semS),
              pltpu.async_copy(w0_v.at[pl.ds(H, H)], wrow_hbm.at[i0b], semS),
              pltpu.async_copy(w1_v.at[pl.ds(H, H)], wrow_hbm.at[i1b], semS)]
        for c in sa + sb:
            c.wait()

    return k(flat_x, dest, wpair)


def _ffn_body(be_ref, vld_ref, xs_ref, wg_ref, bg_ref, wu_ref, bu_ref,
              wd_ref, bd_ref, wrow_ref, ys_ref):
    b = pl.program_id(0)

    @pl.when(vld_ref[b] == 1)
    def _compute():
        x = xs_ref[...]
        g = jnp.dot(x, wg_ref[0], preferred_element_type=jnp.float32) + bg_ref[0]
        u = jnp.dot(x, wu_ref[0], preferred_element_type=jnp.float32) + bu_ref[0]
        h = g * jax.nn.sigmoid(g) * u
        y = jnp.dot(h, wd_ref[0], preferred_element_type=jnp.float32) + bd_ref[0]
        ys_ref[...] = y * wrow_ref[0]


def _ffn(xs, wg, bg, wu, bu, wd, bd, wrow3, be, vld, *, interpret=False):
    grid_spec = pltpu.PrefetchScalarGridSpec(
        num_scalar_prefetch=2,
        grid=(NB,),
        in_specs=[
            pl.BlockSpec((BLK, D), lambda b, be, vld: (b, 0)),
            pl.BlockSpec((1, D, F), lambda b, be, vld: (be[b], 0, 0)),
            pl.BlockSpec((1, 1, F), lambda b, be, vld: (be[b], 0, 0)),
            pl.BlockSpec((1, D, F), lambda b, be, vld: (be[b], 0, 0)),
            pl.BlockSpec((1, 1, F), lambda b, be, vld: (be[b], 0, 0)),
            pl.BlockSpec((1, F, D), lambda b, be, vld: (be[b], 0, 0)),
            pl.BlockSpec((1, 1, D), lambda b, be, vld: (be[b], 0, 0)),
            pl.BlockSpec((1, BLK, 1), lambda b, be, vld: (b, 0, 0)),
        ],
        out_specs=pl.BlockSpec((BLK, D), lambda b, be, vld: (b, 0)),
    )
    return pl.pallas_call(
        _ffn_body,
        grid_spec=grid_spec,
        out_shape=jax.ShapeDtypeStruct((ROWS, D), jnp.float32),
        compiler_params=pltpu.CompilerParams(
            vmem_limit_bytes=120 * 1024 * 1024),
        interpret=interpret,
    )(be, vld, xs, wg, bg, wu, bu, wd, bd, wrow3)


def _combine_sc(ys, dest):
    mesh = plsc.VectorSubcoreMesh(core_axis_name="c", subcore_axis_name="s")
    nch = TPW // CHUNK

    @functools.partial(
        pl.kernel,
        out_type=jax.ShapeDtypeStruct((S, D), jnp.float32),
        mesh=mesh,
        scratch_types=[
            pltpu.VMEM((CHUNK,), jnp.int32),
            pltpu.VMEM((CHUNK,), jnp.int32),
            pltpu.VMEM((CHUNK, D), jnp.float32),
            pltpu.VMEM((CHUNK, D), jnp.float32),
            pltpu.VMEM((CHUNK, D), jnp.float32),
            pltpu.SemaphoreType.DMA,
        ],
    )
    def k(ys_hbm, dest_hbm, out_hbm, i0_v, i1_v, a_v, b_v, o_v, sem):
        wid = lax.axis_index("s") * NC + lax.axis_index("c")
        base = wid * TPW

        def chunk(c, _):
            cb = base + c * CHUNK
            pltpu.sync_copy(dest_hbm.at[pl.ds(cb, CHUNK)], i0_v)
            pltpu.sync_copy(dest_hbm.at[pl.ds(S + cb, CHUNK)], i1_v)
            g0 = pltpu.async_copy(ys_hbm.at[i0_v], a_v, sem)
            g1 = pltpu.async_copy(ys_hbm.at[i1_v], b_v, sem)
            g0.wait()
            g1.wait()

            def tok(i, _):
                for v in range(D // 16):
                    sl = pl.ds(v * 16, 16)
                    o_v[i, sl] = a_v[i, sl] + b_v[i, sl]
                return 0

            lax.fori_loop(0, CHUNK, tok, 0)
            pltpu.sync_copy(o_v, out_hbm.at[pl.ds(cb, CHUNK)])
            return 0

        lax.fori_loop(0, nch, chunk, 0)

    return k(ys, dest)


def kernel(x, router_w, router_b, wg, bg, wu, bu, wd, bd):
    bsz, seq, _ = x.shape
    flat_x = x.reshape(S, D)
    rw_pad = jnp.zeros((D, EP), jnp.float32).at[:, :E].set(router_w)
    rb_padT = jnp.zeros((EP, 1), jnp.float32).at[:E, 0].set(router_b)

    probsT, meta = _router(flat_x, rw_pad, rb_padT)
    i1 = meta[0].astype(jnp.int32)
    i2 = meta[1].astype(jnp.int32)

    e_cm = jnp.concatenate([meta[0].reshape(E, BLK).T,
                            meta[1].reshape(E, BLK).T], axis=1)
    dest2d, bev = _dispatch(e_cm)
    dest = dest2d.T.reshape(P)
    be = bev[:NB, 0]
    vld = bev[:NB, 1]

    wpair = jnp.concatenate([meta[2], meta[3]])
    xs, wrow = _scatter_sc(flat_x, dest, wpair)
    ys = _ffn(xs, wg, bg.reshape(E, 1, F), wu, bu.reshape(E, 1, F),
              wd, bd.reshape(E, 1, D), wrow.reshape(NB, BLK, 1), be, vld)
    out = _combine_sc(ys, dest)

    return (out.reshape(bsz, seq, D),
            probsT.T.reshape(bsz, seq, E),
            jnp.stack([i1, i2], axis=1).reshape(bsz, seq, K))
